# baseline (speedup 1.0000x reference)
import functools

import jax
import jax.numpy as jnp
from jax import lax
from jax.experimental import pallas as pl
from jax.experimental.pallas import tpu as pltpu
from jax.experimental.pallas import tpu_sc as plsc

_NC = 2
_NS = 16
_NW = _NC * _NS
_LANES = 16


def _matmul_stage(table, W_pad, inv_seq):
  V, E = table.shape
  DP = W_pad.shape[1]
  BLK = 8000
  assert V % BLK == 0

  def body(t_ref, w_ref, o_ref):
    o_ref[...] = jnp.dot(
        t_ref[...], w_ref[...], preferred_element_type=jnp.float32
    ) * inv_seq

  return pl.pallas_call(
      body,
      grid=(V // BLK,),
      in_specs=[
          pl.BlockSpec((BLK, E), lambda i: (i, 0)),
          pl.BlockSpec((E, DP), lambda i: (0, 0)),
      ],
      out_specs=pl.BlockSpec((BLK, DP), lambda i: (i, 0)),
      out_shape=jax.ShapeDtypeStruct((V, DP), jnp.float32),
  )(table, W_pad)


def _pool_stage(x2, tableW, b_pad, B, S):
  DP = tableW.shape[1]
  H = S // 2
  CB = 8
  RPC = 2 * CB
  per_w = B // _NW
  n_chunks = per_w // CB

  mesh = plsc.VectorSubcoreMesh(core_axis_name="c", subcore_axis_name="s")

  @functools.partial(
      pl.kernel,
      out_type=jax.ShapeDtypeStruct((B, DP), jnp.float32),
      mesh=mesh,
      scratch_types=[
          pltpu.VMEM((RPC, H), jnp.int32),
          pltpu.VMEM((RPC, H, DP), jnp.float32),
          pltpu.VMEM((CB, DP), jnp.float32),
          pltpu.VMEM((DP,), jnp.float32),
          pltpu.SemaphoreType.DMA,
      ],
      compiler_params=pltpu.CompilerParams(use_tc_tiling_on_sc=False),
  )
  def body(x2_hbm, tw_hbm, b_hbm, out_hbm, idx_v, rows_v, out_v, b_v, sem):
    wid = lax.axis_index("s") * _NC + lax.axis_index("c")
    pltpu.sync_copy(b_hbm, b_v)

    def chunk_body(ci, carry):
      r0 = wid * (2 * per_w) + ci * RPC
      pltpu.sync_copy(x2_hbm.at[pl.ds(r0, RPC)], idx_v)
      cps = [
          pltpu.async_copy(tw_hbm.at[idx_v.at[j]], rows_v.at[j], sem)
          for j in range(RPC)
      ]
      for cp in cps:
        cp.wait()
      bvec = b_v[...]
      for e in range(CB):
        def tok_body(t, acc):
          return acc + rows_v[2 * e, t, :] + rows_v[2 * e + 1, t, :]
        accv = lax.fori_loop(
            0, H, tok_body, jnp.zeros((DP,), jnp.float32)
        )
        out_v[e, :] = accv + bvec
      pltpu.sync_copy(out_v, out_hbm.at[pl.ds(wid * per_w + ci * CB, CB)])
      return carry

    lax.fori_loop(0, n_chunks, chunk_body, 0)

  return body(x2, tableW, b_pad)


def kernel(x, table, W, b):
  B, S = x.shape
  V, E = table.shape
  C = W.shape[1]
  DP = _LANES

  W_pad = jnp.concatenate([W, jnp.zeros((E, DP - C), W.dtype)], axis=1)
  b_pad = jnp.concatenate([b, jnp.zeros((DP - C,), b.dtype)])
  x2 = x.astype(jnp.int32).reshape(B * 2, S // 2)

  tableW = _matmul_stage(table, W_pad, 1.0 / S)
  out_pad = _pool_stage(x2, tableW, b_pad, B, S)
  return out_pad[:, :C]


# speedup vs baseline: 3.5757x; 3.5757x over previous
import functools

import jax
import jax.numpy as jnp
from jax import lax
from jax.experimental import pallas as pl
from jax.experimental.pallas import tpu as pltpu
from jax.experimental.pallas import tpu_sc as plsc

_NC = 2
_NS = 16
_NW = _NC * _NS
_LANES = 16


def _pool_stage(x2, table):
  V, E = table.shape
  B2, H = x2.shape
  B = B2 // 2
  KC = E // _LANES
  per_w = B // _NW

  mesh = plsc.VectorSubcoreMesh(core_axis_name="c", subcore_axis_name="s")

  @functools.partial(
      pl.kernel,
      out_type=jax.ShapeDtypeStruct((B, E), jnp.float32),
      mesh=mesh,
      scratch_types=[
          pltpu.VMEM((2 * per_w, H), jnp.int32),
          pltpu.VMEM((2, 2, H, E), jnp.float32),
          pltpu.VMEM((8, E), jnp.float32),
          pltpu.SemaphoreType.DMA,
          pltpu.SemaphoreType.DMA,
      ],
  )
  def body(x2_hbm, tab_hbm, out_hbm, idx_v, rows_v, out_v, sem0, sem1):
    wid = lax.axis_index("s") * _NC + lax.axis_index("c")
    sems = (sem0, sem1)
    pltpu.sync_copy(
        x2_hbm.at[pl.ds(pl.multiple_of(wid * 2 * per_w, 8), 2 * per_w)],
        idx_v,
    )

    def fire(ei, slot):
      for j in range(2):
        pltpu.async_copy(
            tab_hbm.at[idx_v.at[2 * ei + j]], rows_v.at[slot, j], sems[slot]
        )

    def drain(slot):
      for j in range(2):
        pltpu.make_async_copy(
            tab_hbm.at[idx_v.at[0]], rows_v.at[slot, j], sems[slot]
        ).wait()

    def run(ci0, carry):
      for d in range(2):
        slot = d
        @pl.when(ci0 + d + 1 < per_w)
        def _():
          fire(ci0 + d + 1, 1 - slot)
        drain(slot)
        acc = tuple(
            jnp.zeros((_LANES,), jnp.float32) for _ in range(KC)
        )
        for j in range(2):
          def g_body(g, acc):
            out = list(acc)
            for u in range(4):
              t = 4 * g + u
              for k in range(KC):
                out[k] = out[k] + rows_v[slot, j, t, pl.ds(k * _LANES, _LANES)]
            return tuple(out)
          acc = lax.fori_loop(0, H // 4, g_body, acc)
        row = (ci0 % 8) + d
        for k in range(KC):
          out_v[row, pl.ds(k * _LANES, _LANES)] = acc[k]
        if d == 1:
          @pl.when(ci0 % 8 == 6)
          def _():
            pltpu.sync_copy(
                out_v,
                out_hbm.at[pl.ds(pl.multiple_of(wid * per_w + ci0 - 6, 8), 8)],
            )
      return carry

    fire(0, 0)
    lax.fori_loop(0, per_w // 2, lambda i, c: run(2 * i, c), 0)

  return body(x2, table)


def _head_stage(pooled, W_pad, b_pad, inv_seq):
  B, E = pooled.shape
  DP = W_pad.shape[1]

  def mm_body(p_ref, w_ref, b_ref, o_ref):
    o_ref[...] = (
        jnp.dot(p_ref[...], w_ref[...], preferred_element_type=jnp.float32)
        * inv_seq
        + b_ref[...]
    )

  return pl.pallas_call(
      mm_body,
      out_shape=jax.ShapeDtypeStruct((B, DP), jnp.float32),
  )(pooled, W_pad, b_pad.reshape(1, DP))


def kernel(x, table, W, b):
  B, S = x.shape
  E = table.shape[1]
  C = W.shape[1]
  DP = 8

  W_pad = jnp.concatenate([W, jnp.zeros((E, DP - C), W.dtype)], axis=1)
  b_pad = jnp.concatenate([b, jnp.zeros((DP - C,), b.dtype)])
  x2 = x.astype(jnp.int32).reshape(B * 2, S // 2)

  pooled = _pool_stage(x2, table)
  out_pad = _head_stage(pooled, W_pad, b_pad, 1.0 / S)
  return out_pad[:, :C]


# speedup vs baseline: 3.5891x; 1.0037x over previous
import functools

import jax
import jax.numpy as jnp
from jax import lax
from jax.experimental import pallas as pl
from jax.experimental.pallas import tpu as pltpu
from jax.experimental.pallas import tpu_sc as plsc

_NC = 2
_NS = 16
_NW = _NC * _NS
_LANES = 16


def _pool_stage(x2, table):
  V, E = table.shape
  B2, H = x2.shape
  B = B2 // 2
  KC = E // _LANES
  per_w = B // _NW

  mesh = plsc.VectorSubcoreMesh(core_axis_name="c", subcore_axis_name="s")

  @functools.partial(
      pl.kernel,
      out_type=jax.ShapeDtypeStruct((B, E), jnp.float32),
      mesh=mesh,
      scratch_types=[
          pltpu.VMEM((2 * per_w, H), jnp.int32),
          pltpu.VMEM((2, 2, H, E), jnp.float32),
          pltpu.VMEM((2, 8, E), jnp.float32),
          pltpu.SemaphoreType.DMA,
          pltpu.SemaphoreType.DMA,
          pltpu.SemaphoreType.DMA,
      ],
  )
  def body(
      x2_hbm, tab_hbm, out_hbm, idx_v, rows_v, out_v, sem0, sem1, sem_out
  ):
    wid = lax.axis_index("s") * _NC + lax.axis_index("c")
    sems = (sem0, sem1)
    pltpu.sync_copy(
        x2_hbm.at[pl.ds(pl.multiple_of(wid * 2 * per_w, 8), 2 * per_w)],
        idx_v,
    )

    def fire(ei, slot):
      for j in range(2):
        pltpu.async_copy(
            tab_hbm.at[idx_v.at[2 * ei + j]], rows_v.at[slot, j], sems[slot]
        )

    def drain(slot):
      for j in range(2):
        pltpu.make_async_copy(
            tab_hbm.at[idx_v.at[0]], rows_v.at[slot, j], sems[slot]
        ).wait()

    def drain_out():
      pltpu.make_async_copy(
          out_v.at[0],
          out_hbm.at[pl.ds(pl.multiple_of(wid * per_w, 8), 8)],
          sem_out,
      ).wait()

    def run(ci0, carry):
      for d in range(2):
        slot = d
        @pl.when(ci0 + d + 1 < per_w)
        def _():
          fire(ci0 + d + 1, 1 - slot)
        if d == 0:
          @pl.when((ci0 % 8 == 0) & (ci0 >= 16))
          def _():
            drain_out()
        drain(slot)
        acc = tuple(
            jnp.zeros((_LANES,), jnp.float32) for _ in range(KC)
        )
        def g_body(g, acc):
          out = list(acc)
          for u in range(4):
            t = 4 * g + u
            for j in range(2):
              for k in range(KC):
                out[k] = out[k] + rows_v[slot, j, t, pl.ds(k * _LANES, _LANES)]
          return tuple(out)
        acc = lax.fori_loop(0, H // 4, g_body, acc)
        buf = (ci0 // 8) % 2
        row = (ci0 % 8) + d
        for k in range(KC):
          out_v[buf, row, pl.ds(k * _LANES, _LANES)] = acc[k]
        if d == 1:
          @pl.when(ci0 % 8 == 6)
          def _():
            pltpu.async_copy(
                out_v.at[buf],
                out_hbm.at[pl.ds(pl.multiple_of(wid * per_w + ci0 - 6, 8), 8)],
                sem_out,
            )
      return carry

    fire(0, 0)
    lax.fori_loop(0, per_w // 2, lambda i, c: run(2 * i, c), 0)
    drain_out()
    drain_out()

  return body(x2, table)


def _head_stage(pooled, W_pad, b_pad, inv_seq):
  B, E = pooled.shape
  DP = W_pad.shape[1]

  def mm_body(p_ref, w_ref, b_ref, o_ref):
    o_ref[...] = (
        jnp.dot(p_ref[...], w_ref[...], preferred_element_type=jnp.float32)
        * inv_seq
        + b_ref[...]
    )

  return pl.pallas_call(
      mm_body,
      out_shape=jax.ShapeDtypeStruct((B, DP), jnp.float32),
  )(pooled, W_pad, b_pad.reshape(1, DP))


def kernel(x, table, W, b):
  B, S = x.shape
  E = table.shape[1]
  C = W.shape[1]
  DP = 8

  W_pad = jnp.concatenate([W, jnp.zeros((E, DP - C), W.dtype)], axis=1)
  b_pad = jnp.concatenate([b, jnp.zeros((DP - C,), b.dtype)])
  x2 = x.astype(jnp.int32).reshape(B * 2, S // 2)

  pooled = _pool_stage(x2, table)
  out_pad = _head_stage(pooled, W_pad, b_pad, 1.0 / S)
  return out_pad[:, :C]


# speedup vs baseline: 4.3580x; 1.2142x over previous
import functools

import jax
import jax.numpy as jnp
from jax import lax
from jax.experimental import pallas as pl
from jax.experimental.pallas import tpu as pltpu
from jax.experimental.pallas import tpu_sc as plsc

_NC = 2
_NS = 16
_NW = _NC * _NS
_LANES = 16


def _pool_stage(x2, table):
  V, E = table.shape
  B4, H = x2.shape
  B = B4 // 4
  KC = E // _LANES
  per_w = B // _NW

  mesh = plsc.VectorSubcoreMesh(core_axis_name="c", subcore_axis_name="s")

  @functools.partial(
      pl.kernel,
      out_type=jax.ShapeDtypeStruct((B, E), jnp.float32),
      mesh=mesh,
      scratch_types=[
          pltpu.VMEM((4 * per_w, H), jnp.int32),
          pltpu.VMEM((8, H, E), jnp.float32),
          pltpu.VMEM((2, 8, E), jnp.float32),
          [pltpu.SemaphoreType.DMA] * 8,
          pltpu.SemaphoreType.DMA,
      ],
  )
  def body(
      x2_hbm, tab_hbm, out_hbm, idx_v, rows_v, out_v, sems, sem_out
  ):
    wid = lax.axis_index("s") * _NC + lax.axis_index("c")
    NJ = 4 * per_w
    pltpu.sync_copy(
        x2_hbm.at[pl.ds(pl.multiple_of(wid * NJ, 8), NJ)],
        idx_v,
    )

    def fire(j, slot):
      pltpu.async_copy(tab_hbm.at[idx_v.at[j]], rows_v.at[slot], sems[slot])

    def drain(slot):
      pltpu.make_async_copy(
          tab_hbm.at[idx_v.at[0]], rows_v.at[slot], sems[slot]
      ).wait()

    def drain_out():
      pltpu.make_async_copy(
          out_v.at[0],
          out_hbm.at[pl.ds(pl.multiple_of(wid * per_w, 8), 8)],
          sem_out,
      ).wait()

    def partial_reduce(slot, acc):
      def g_body(g, acc):
        out = list(acc)
        for u in range(5):
          t = 5 * g + u
          for k in range(KC):
            out[k] = out[k] + rows_v[slot, t, pl.ds(k * _LANES, _LANES)]
        return tuple(out)
      return lax.fori_loop(0, H // 5, g_body, acc)

    def finalize_elem(e, acc):
      @pl.when((e % 8 == 0) & (e >= 16))
      def _():
        drain_out()
      buf = (e // 8) % 2
      row = e % 8
      for k in range(KC):
        out_v[buf, row, pl.ds(k * _LANES, _LANES)] = acc[k]
      @pl.when(e % 8 == 7)
      def _():
        pltpu.async_copy(
            out_v.at[buf],
            out_hbm.at[pl.ds(pl.multiple_of(wid * per_w + e - 7, 8), 8)],
            sem_out,
        )

    def run(j0, carry):
      for half in range(2):
        acc = tuple(jnp.zeros((_LANES,), jnp.float32) for _ in range(KC))
        for q in range(4):
          d = 4 * half + q
          jcur = j0 + d
          @pl.when(jcur + 6 < NJ)
          def _():
            fire(jcur + 6, (d + 6) % 8)
          drain(d)
          acc = partial_reduce(d, acc)
        finalize_elem(j0 // 4 + half, acc)
      return carry

    for j in range(6):
      fire(j, j)
    lax.fori_loop(0, NJ // 8, lambda i, c: run(8 * i, c), 0)
    drain_out()
    drain_out()

  return body(x2, table)


def _head_stage(pooled, W_pad, b_pad, inv_seq):
  B, E = pooled.shape
  DP = W_pad.shape[1]

  def mm_body(p_ref, w_ref, b_ref, o_ref):
    o_ref[...] = (
        jnp.dot(p_ref[...], w_ref[...], preferred_element_type=jnp.float32)
        * inv_seq
        + b_ref[...]
    )

  return pl.pallas_call(
      mm_body,
      out_shape=jax.ShapeDtypeStruct((B, DP), jnp.float32),
  )(pooled, W_pad, b_pad.reshape(1, DP))


def kernel(x, table, W, b):
  B, S = x.shape
  E = table.shape[1]
  C = W.shape[1]
  DP = 8

  W_pad = jnp.concatenate([W, jnp.zeros((E, DP - C), W.dtype)], axis=1)
  b_pad = jnp.concatenate([b, jnp.zeros((DP - C,), b.dtype)])
  x2 = x.astype(jnp.int32).reshape(B * 4, S // 4)

  pooled = _pool_stage(x2, table)
  out_pad = _head_stage(pooled, W_pad, b_pad, 1.0 / S)
  return out_pad[:, :C]


# speedup vs baseline: 4.3752x; 1.0039x over previous
import functools

import jax
import jax.numpy as jnp
from jax import lax
from jax.experimental import pallas as pl
from jax.experimental.pallas import tpu as pltpu
from jax.experimental.pallas import tpu_sc as plsc

_NC = 2
_NS = 16
_NW = _NC * _NS
_LANES = 16
_RPE = 4


def _pool_stage(x2, table, Wt, b, inv_seq, DP):
  V, E = table.shape
  C = Wt.shape[0]
  BR, H = x2.shape
  B = BR // _RPE
  KC = E // _LANES
  per_w = B // _NW
  NSLOT = 2 * _RPE
  DIST = NSLOT - 2

  mesh = plsc.VectorSubcoreMesh(core_axis_name="c", subcore_axis_name="s")

  @functools.partial(
      pl.kernel,
      out_type=jax.ShapeDtypeStruct((B, DP), jnp.float32),
      mesh=mesh,
      scratch_types=[
          pltpu.VMEM((_RPE * per_w, H), jnp.int32),
          pltpu.VMEM((NSLOT, H, E), jnp.float32),
          pltpu.VMEM((2, 8, DP), jnp.float32),
          pltpu.VMEM((C, E), jnp.float32),
          pltpu.VMEM((_LANES,), jnp.float32),
          [pltpu.SemaphoreType.DMA] * NSLOT,
          pltpu.SemaphoreType.DMA,
      ],
  )
  def body(
      x2_hbm, tab_hbm, wt_hbm, b_hbm, out_hbm,
      idx_v, rows_v, out_v, w_v, b_v, sems, sem_out,
  ):
    wid = lax.axis_index("s") * _NC + lax.axis_index("c")
    NJ = _RPE * per_w
    pltpu.sync_copy(
        x2_hbm.at[pl.ds(pl.multiple_of(wid * NJ, 8), NJ)],
        idx_v,
    )
    pltpu.sync_copy(wt_hbm, w_v)
    pltpu.sync_copy(b_hbm, b_v)
    bvals = b_v[...]
    wvec = [
        [w_v[c, pl.ds(k * _LANES, _LANES)] for k in range(KC)]
        for c in range(C)
    ]

    def fire(j, slot):
      pltpu.async_copy(tab_hbm.at[idx_v.at[j]], rows_v.at[slot], sems[slot])

    def drain(slot):
      pltpu.make_async_copy(
          tab_hbm.at[idx_v.at[0]], rows_v.at[slot], sems[slot]
      ).wait()

    def drain_out():
      pltpu.make_async_copy(
          out_v.at[0],
          out_hbm.at[pl.ds(pl.multiple_of(wid * per_w, 8), 8)],
          sem_out,
      ).wait()

    def partial_reduce(slot, acc):
      def g_body(g, acc):
        out = list(acc)
        for u in range(5):
          t = 5 * g + u
          for k in range(KC):
            out[k] = out[k] + rows_v[slot, t, pl.ds(k * _LANES, _LANES)]
        return tuple(out)
      return lax.fori_loop(0, H // 5, g_body, acc)

    def finalize_elem(e, acc):
      @pl.when((e % 8 == 0) & (e >= 16))
      def _():
        drain_out()
      buf = (e // 8) % 2
      row = e % 8
      lanes = lax.iota(jnp.int32, _LANES)
      logit_vec = jnp.zeros((_LANES,), jnp.float32)
      for c in range(C):
        m = acc[0] * wvec[c][0]
        for k in range(1, KC):
          m = m + acc[k] * wvec[c][k]
        for fold in (8, 4, 2, 1):
          perm = jnp.bitwise_xor(lanes, fold)
          m = m + m.at[perm].get(mode="promise_in_bounds")
        o = m * inv_seq + bvals[c]
        logit_vec = jnp.where(lanes == c, o, logit_vec)
      out_v[buf, row, :] = logit_vec
      @pl.when(e % 8 == 7)
      def _():
        pltpu.async_copy(
            out_v.at[buf],
            out_hbm.at[pl.ds(pl.multiple_of(wid * per_w + e - 7, 8), 8)],
            sem_out,
        )

    def run(j0, carry):
      for half in range(2):
        acc = tuple(jnp.zeros((_LANES,), jnp.float32) for _ in range(KC))
        for q in range(_RPE):
          d = _RPE * half + q
          jcur = j0 + d
          @pl.when(jcur + DIST < NJ)
          def _():
            fire(jcur + DIST, (d + DIST) % NSLOT)
          drain(d)
          acc = partial_reduce(d, acc)
        finalize_elem(j0 // _RPE + half, acc)
      return carry

    for j in range(DIST):
      fire(j, j)
    lax.fori_loop(0, NJ // NSLOT, lambda i, c: run(NSLOT * i, c), 0)
    drain_out()
    drain_out()

  return body(x2, table, Wt, b)


def kernel(x, table, W, b):
  B, S = x.shape
  C = W.shape[1]
  DP = _LANES

  x2 = x.astype(jnp.int32).reshape(B * _RPE, S // _RPE)
  b_pad = jnp.concatenate([b, jnp.zeros((_LANES - C,), b.dtype)])
  out_pad = _pool_stage(x2, table, W.T, b_pad, 1.0 / S, DP)
  return out_pad[:, :C]
